# baseline (speedup 1.0000x reference)
import jax
import jax.numpy as jnp
from jax.experimental import pallas as pl
from jax.experimental.pallas import tpu as pltpu

_DIM = 768
_NUM_OUT = 64
_BM = 2048
_NBUF = 3

_NEG_INF = float("-inf")


def _router_block(x_hbm, w_ref, b_ref, probs_ref, idx_ref, xbuf, sem):
    i = pl.program_id(0)
    n_steps = pl.num_programs(0)

    def copy(step, slot):
        return pltpu.make_async_copy(
            x_hbm.at[pl.ds(step * _BM, _BM), :], xbuf.at[slot], sem.at[slot]
        )

    @pl.when(i == 0)
    def _prologue():
        for k in range(_NBUF):
            copy(k, k).start()

    slot = jax.lax.rem(i, _NBUF)
    copy(i, slot).wait()

    x = xbuf[slot]
    w = w_ref[...]
    logits = jax.lax.dot_general(
        w, x, (((1,), (1,)), ((), ())), preferred_element_type=jnp.float32
    )
    logits = logits + b_ref[...]

    iota = jax.lax.broadcasted_iota(jnp.int32, logits.shape, 0).astype(jnp.float32)
    big = float(_NUM_OUT)

    v1 = jnp.max(logits, axis=0, keepdims=True)
    i1f = jnp.min(jnp.where(logits == v1, iota, big), axis=0, keepdims=True)
    masked = jnp.where(iota == i1f, _NEG_INF, logits)
    v2 = jnp.max(masked, axis=0, keepdims=True)
    i2f = jnp.min(jnp.where(masked == v2, iota, big), axis=0, keepdims=True)

    t = jnp.exp(v2 - v1)
    denom = 1.0 + t
    probs_ref[...] = jnp.concatenate([1.0 / denom, t / denom], axis=0)
    idx_ref[...] = jnp.concatenate(
        [i1f.astype(jnp.int32), i2f.astype(jnp.int32)], axis=0
    )

    @pl.when(i + _NBUF < n_steps)
    def _prefetch():
        copy(i + _NBUF, slot).start()


def kernel(input, W, b):
    n_tok = input.shape[0]
    grid = (n_tok // _BM,)
    b2d = b.reshape(_NUM_OUT, 1)
    probs_t, idx_t = pl.pallas_call(
        _router_block,
        grid=grid,
        in_specs=[
            pl.BlockSpec(memory_space=pltpu.HBM),
            pl.BlockSpec((_NUM_OUT, _DIM), lambda i: (0, 0)),
            pl.BlockSpec((_NUM_OUT, 1), lambda i: (0, 0)),
        ],
        out_specs=[
            pl.BlockSpec((2, _BM), lambda i: (0, i)),
            pl.BlockSpec((2, _BM), lambda i: (0, i)),
        ],
        out_shape=[
            jax.ShapeDtypeStruct((2, n_tok), jnp.float32),
            jax.ShapeDtypeStruct((2, n_tok), jnp.int32),
        ],
        scratch_shapes=[
            pltpu.VMEM((_NBUF, _BM, _DIM), jnp.float32),
            pltpu.SemaphoreType.DMA((_NBUF,)),
        ],
        compiler_params=pltpu.CompilerParams(
            dimension_semantics=("arbitrary",),
        ),
    )(input, W, b2d)
    return probs_t.T, idx_t.T


# speedup vs baseline: 1.0285x; 1.0285x over previous
import jax
import jax.numpy as jnp
from jax.experimental import pallas as pl
from jax.experimental.pallas import tpu as pltpu

_DIM = 768
_NUM_OUT = 64
_BM = 1024
_NBUF = 4

_NEG_INF = float("-inf")


def _router_block(x_hbm, w_ref, b_ref, probs_ref, idx_ref, xbuf, sem):
    i = pl.program_id(0)
    n_steps = pl.num_programs(0)

    def copy(step, slot):
        return pltpu.make_async_copy(
            x_hbm.at[pl.ds(step * _BM, _BM), :], xbuf.at[slot], sem.at[slot]
        )

    @pl.when(i == 0)
    def _prologue():
        for k in range(_NBUF):
            copy(k, k).start()

    slot = jax.lax.rem(i, _NBUF)
    copy(i, slot).wait()

    x = xbuf[slot]
    w = w_ref[...]
    logits = jax.lax.dot_general(
        w, x, (((1,), (1,)), ((), ())), preferred_element_type=jnp.float32
    )
    logits = logits + b_ref[...]

    iota = jax.lax.broadcasted_iota(jnp.int32, logits.shape, 0).astype(jnp.float32)
    big = float(_NUM_OUT)

    v1 = jnp.max(logits, axis=0, keepdims=True)
    i1f = jnp.min(jnp.where(logits == v1, iota, big), axis=0, keepdims=True)
    masked = jnp.where(iota == i1f, _NEG_INF, logits)
    v2 = jnp.max(masked, axis=0, keepdims=True)
    i2f = jnp.min(jnp.where(masked == v2, iota, big), axis=0, keepdims=True)

    t = jnp.exp(v2 - v1)
    denom = 1.0 + t
    probs_ref[...] = jnp.concatenate([1.0 / denom, t / denom], axis=0)
    idx_ref[...] = jnp.concatenate(
        [i1f.astype(jnp.int32), i2f.astype(jnp.int32)], axis=0
    )

    @pl.when(i + _NBUF < n_steps)
    def _prefetch():
        copy(i + _NBUF, slot).start()


def kernel(input, W, b):
    n_tok = input.shape[0]
    grid = (n_tok // _BM,)
    b2d = b.reshape(_NUM_OUT, 1)
    probs_t, idx_t = pl.pallas_call(
        _router_block,
        grid=grid,
        in_specs=[
            pl.BlockSpec(memory_space=pltpu.HBM),
            pl.BlockSpec((_NUM_OUT, _DIM), lambda i: (0, 0)),
            pl.BlockSpec((_NUM_OUT, 1), lambda i: (0, 0)),
        ],
        out_specs=[
            pl.BlockSpec((2, _BM), lambda i: (0, i)),
            pl.BlockSpec((2, _BM), lambda i: (0, i)),
        ],
        out_shape=[
            jax.ShapeDtypeStruct((2, n_tok), jnp.float32),
            jax.ShapeDtypeStruct((2, n_tok), jnp.int32),
        ],
        scratch_shapes=[
            pltpu.VMEM((_NBUF, _BM, _DIM), jnp.float32),
            pltpu.SemaphoreType.DMA((_NBUF,)),
        ],
        compiler_params=pltpu.CompilerParams(
            dimension_semantics=("arbitrary",),
        ),
    )(input, W, b2d)
    return probs_t.T, idx_t.T
